# baseline (speedup 1.0000x reference)
import functools

import jax
import jax.numpy as jnp
from jax import lax
from jax.experimental import pallas as pl
from jax.experimental.pallas import tpu as pltpu
from jax.experimental.pallas import tpu_sc as plsc

N = 320000
D = 128
NUM_SEGMENTS = 10000


TC_B = 6400


K_AUG = 144


def _tc_body(z_ref, post_ref, w1t_ref, embpt_ref, b1c_ref, wpt_ref, wout_ref,
             bout_ref, y_ref, lhshi_s, lhslo_s, rhs_s):
    @pl.when(pl.program_id(0) == 0)
    def _fold_weights():
        tt = jnp.dot(w1t_ref[...], embpt_ref[...],
                     preferred_element_type=jnp.float32) + b1c_ref[...]
        mt = jnp.dot(w1t_ref[...], wpt_ref[...],
                     preferred_element_type=jnp.float32)
        tthi = tt.astype(jnp.bfloat16)
        mthi = mt.astype(jnp.bfloat16)
        lhshi_s[:, 0:D] = tthi
        lhshi_s[:, D:D + 3] = mthi
        lhshi_s[:, D + 3:K_AUG] = jnp.zeros((D, K_AUG - D - 3), jnp.bfloat16)
        lhslo_s[:, 0:D] = (tt - tthi.astype(jnp.float32)).astype(jnp.bfloat16)
        lhslo_s[:, D:D + 3] = (mt - mthi.astype(jnp.float32)).astype(
            jnp.bfloat16)
        lhslo_s[:, D + 3:K_AUG] = jnp.zeros((D, K_AUG - D - 3), jnp.bfloat16)
        rhs_s[D + 3:K_AUG, :] = jnp.zeros((K_AUG - D - 3, TC_B), jnp.bfloat16)

    z = z_ref[...]
    oht = (lax.broadcasted_iota(jnp.int32, (D, TC_B), 0) == z)
    rhs_s[0:D, :] = oht.astype(jnp.bfloat16)
    rhs_s[D:D + 3, :] = post_ref[...].astype(jnp.bfloat16)
    r = rhs_s[...]
    at = (jnp.dot(lhshi_s[...], r, preferred_element_type=jnp.float32)
          + jnp.dot(lhslo_s[...], r, preferred_element_type=jnp.float32))
    ht = jnp.tanh(at)
    y_ref[...] = (jnp.dot(wout_ref[...], ht,
                          preferred_element_type=jnp.float32)
                  + bout_ref[...])


def _tc_energies(z1, post, w1t, embpt, b1c, wpt, wout, boutr):
    grid = (N // TC_B,)
    return pl.pallas_call(
        _tc_body,
        grid=grid,
        in_specs=[
            pl.BlockSpec((1, TC_B), lambda i: (0, i)),
            pl.BlockSpec((3, TC_B), lambda i: (0, i)),
            pl.BlockSpec((D, D), lambda i: (0, 0)),
            pl.BlockSpec((D, D), lambda i: (0, 0)),
            pl.BlockSpec((D, 1), lambda i: (0, 0)),
            pl.BlockSpec((D, 3), lambda i: (0, 0)),
            pl.BlockSpec((1, D), lambda i: (0, 0)),
            pl.BlockSpec((1, 1), lambda i: (0, 0)),
        ],
        out_specs=pl.BlockSpec((1, TC_B), lambda i: (0, i)),
        out_shape=jax.ShapeDtypeStruct((1, N), jnp.float32),
        scratch_shapes=[
            pltpu.VMEM((D, K_AUG), jnp.bfloat16),
            pltpu.VMEM((D, K_AUG), jnp.bfloat16),
            pltpu.VMEM((K_AUG, TC_B), jnp.bfloat16),
        ],
    )(z1, post, w1t, embpt, b1c, wpt, wout, boutr)


NW = 32
CH = N // NW
S_PAD = 10240
SLICE = S_PAD // 16

@functools.lru_cache(maxsize=1)
def _make_sc_segment_sum():
    mesh = plsc.VectorSubcoreMesh(core_axis_name="c", subcore_axis_name="s")
    return pl.kernel(
        _sc_segment_sum_body,
        mesh=mesh,
        compiler_params=pltpu.CompilerParams(needs_layout_passes=False),
        out_type=jax.ShapeDtypeStruct((2, S_PAD), jnp.float32),
        scratch_types=[
            pltpu.VMEM((CH,), jnp.int32),
            pltpu.VMEM((CH,), jnp.float32),
            pltpu.VMEM((S_PAD,), jnp.float32),
            pltpu.VMEM((SLICE,), jnp.float32),
            pltpu.VMEM((SLICE,), jnp.float32),
            pltpu.VMEM_SHARED((16, S_PAD), jnp.float32),
        ],
    )


def _sc_segment_sum_body(batch_hbm, y_hbm, out_hbm, b_v, y_v, acc_v, sum_v,
                         tmp_v, shared):
    cid = lax.axis_index("c")
    sid = lax.axis_index("s")
    wid = cid * 16 + sid

    zeros16 = jnp.zeros((16,), jnp.float32)

    def _zero(i, carry):
        for u in range(8):
            acc_v[pl.ds(i * 128 + u * 16, 16)] = zeros16
        return carry

    lax.fori_loop(0, S_PAD // 128, _zero, 0)

    pltpu.sync_copy(batch_hbm.at[pl.ds(wid * CH, CH)], b_v)
    pltpu.sync_copy(y_hbm.at[pl.ds(wid * CH, CH)], y_v)

    def _scatter(i, carry):
        for u in range(5):
            idx = b_v[pl.ds(i * 80 + u * 16, 16)]
            val = y_v[pl.ds(i * 80 + u * 16, 16)]
            plsc.addupdate_scatter(acc_v, [idx], val)
        return carry

    lax.fori_loop(0, CH // 80, _scatter, 0)

    pltpu.sync_copy(acc_v, shared.at[sid])
    plsc.subcore_barrier()

    pltpu.sync_copy(shared.at[0, pl.ds(sid * SLICE, SLICE)], sum_v)
    for t in range(1, 16):
        pltpu.sync_copy(shared.at[t, pl.ds(sid * SLICE, SLICE)], tmp_v)

        def _accum(j, carry):
            for u in range(8):
                sl = pl.ds(j * 128 + u * 16, 16)
                sum_v[sl] = sum_v[sl] + tmp_v[sl]
            return carry

        lax.fori_loop(0, SLICE // 128, _accum, 0)

    pltpu.sync_copy(sum_v, out_hbm.at[cid, pl.ds(sid * SLICE, SLICE)])


def kernel(z, pos, batch, emb, Wp, W1, b1, W_out, b_out):
    z1 = z.astype(jnp.int32).reshape(1, N)
    post = pos.T
    w1t = W1.T
    embpt = jnp.zeros((D, D), jnp.float32).at[:, : emb.shape[0]].set(emb.T)
    b1c = b1.reshape(D, 1)
    wpt = Wp.T
    boutr = b_out.reshape(1, 1)

    y = _tc_energies(z1, post, w1t, embpt, b1c, wpt, W_out.reshape(1, D),
                     boutr)

    parts = _make_sc_segment_sum()(batch.astype(jnp.int32), y.reshape(N))
    out = (parts[0] + parts[1])[:NUM_SEGMENTS].reshape(NUM_SEGMENTS, 1)
    return out


# speedup vs baseline: 1.2221x; 1.2221x over previous
import functools

import jax
import jax.numpy as jnp
from jax import lax
from jax.experimental import pallas as pl
from jax.experimental.pallas import tpu as pltpu
from jax.experimental.pallas import tpu_sc as plsc

N = 320000
D = 128
NUM_SEGMENTS = 10000


TC_B = 12800


K_AUG = 144


def _tc_body(z_ref, post_ref, w1t_ref, embpt_ref, b1c_ref, wpt_ref, wout_ref,
             bout_ref, y_ref, lhshi_s, lhslo_s, rhs_s):
    @pl.when(pl.program_id(0) == 0)
    def _fold_weights():
        tt = jnp.dot(w1t_ref[...], embpt_ref[...],
                     preferred_element_type=jnp.float32) + b1c_ref[...]
        mt = jnp.dot(w1t_ref[...], wpt_ref[...],
                     preferred_element_type=jnp.float32)
        tthi = tt.astype(jnp.bfloat16)
        mthi = mt.astype(jnp.bfloat16)
        lhshi_s[:, 0:D] = tthi
        lhshi_s[:, D:D + 3] = mthi
        lhshi_s[:, D + 3:K_AUG] = jnp.zeros((D, K_AUG - D - 3), jnp.bfloat16)
        lhslo_s[:, 0:D] = (tt - tthi.astype(jnp.float32)).astype(jnp.bfloat16)
        lhslo_s[:, D:D + 3] = (mt - mthi.astype(jnp.float32)).astype(
            jnp.bfloat16)
        lhslo_s[:, D + 3:K_AUG] = jnp.zeros((D, K_AUG - D - 3), jnp.bfloat16)
        rhs_s[D + 3:K_AUG, :] = jnp.zeros((K_AUG - D - 3, TC_B), jnp.bfloat16)

    z = z_ref[...]
    oht = (lax.broadcasted_iota(jnp.int32, (D, TC_B), 0) == z)
    rhs_s[0:D, :] = oht.astype(jnp.bfloat16)
    rhs_s[D:D + 3, :] = post_ref[...].astype(jnp.bfloat16)
    r = rhs_s[...]
    at = jnp.dot(lhshi_s[...], r, preferred_element_type=jnp.float32)
    ht = jnp.tanh(at)
    y_ref[...] = (jnp.dot(wout_ref[...], ht,
                          preferred_element_type=jnp.float32)
                  + bout_ref[...])


def _tc_energies(z1, post, w1t, embpt, b1c, wpt, wout, boutr):
    grid = (N // TC_B,)
    return pl.pallas_call(
        _tc_body,
        grid=grid,
        in_specs=[
            pl.BlockSpec((1, TC_B), lambda i: (0, i)),
            pl.BlockSpec((3, TC_B), lambda i: (0, i)),
            pl.BlockSpec((D, D), lambda i: (0, 0)),
            pl.BlockSpec((D, D), lambda i: (0, 0)),
            pl.BlockSpec((D, 1), lambda i: (0, 0)),
            pl.BlockSpec((D, 3), lambda i: (0, 0)),
            pl.BlockSpec((1, D), lambda i: (0, 0)),
            pl.BlockSpec((1, 1), lambda i: (0, 0)),
        ],
        out_specs=pl.BlockSpec((1, TC_B), lambda i: (0, i)),
        out_shape=jax.ShapeDtypeStruct((1, N), jnp.float32),
        scratch_shapes=[
            pltpu.VMEM((D, K_AUG), jnp.bfloat16),
            pltpu.VMEM((D, K_AUG), jnp.bfloat16),
            pltpu.VMEM((K_AUG, TC_B), jnp.bfloat16),
        ],
    )(z1, post, w1t, embpt, b1c, wpt, wout, boutr)


NW = 32
CH = N // NW
S_PAD = 10240
SLICE = S_PAD // 16

@functools.lru_cache(maxsize=1)
def _make_sc_segment_sum():
    mesh = plsc.VectorSubcoreMesh(core_axis_name="c", subcore_axis_name="s")
    return pl.kernel(
        _sc_segment_sum_body,
        mesh=mesh,
        compiler_params=pltpu.CompilerParams(needs_layout_passes=False),
        out_type=jax.ShapeDtypeStruct((2, S_PAD), jnp.float32),
        scratch_types=[
            pltpu.VMEM((CH,), jnp.int32),
            pltpu.VMEM((CH,), jnp.float32),
            pltpu.VMEM((S_PAD,), jnp.float32),
            pltpu.VMEM((SLICE,), jnp.float32),
            pltpu.VMEM((SLICE,), jnp.float32),
            pltpu.VMEM_SHARED((16, S_PAD), jnp.float32),
        ],
    )


def _sc_segment_sum_body(batch_hbm, y_hbm, out_hbm, b_v, y_v, acc_v, sum_v,
                         tmp_v, shared):
    cid = lax.axis_index("c")
    sid = lax.axis_index("s")
    wid = cid * 16 + sid

    zeros16 = jnp.zeros((16,), jnp.float32)

    def _zero(i, carry):
        for u in range(8):
            acc_v[pl.ds(i * 128 + u * 16, 16)] = zeros16
        return carry

    lax.fori_loop(0, S_PAD // 128, _zero, 0)

    pltpu.sync_copy(batch_hbm.at[pl.ds(wid * CH, CH)], b_v)
    pltpu.sync_copy(y_hbm.at[pl.ds(wid * CH, CH)], y_v)

    def _scatter(i, carry):
        for u in range(5):
            idx = b_v[pl.ds(i * 80 + u * 16, 16)]
            val = y_v[pl.ds(i * 80 + u * 16, 16)]
            plsc.addupdate_scatter(acc_v, [idx], val)
        return carry

    lax.fori_loop(0, CH // 80, _scatter, 0)

    pltpu.sync_copy(acc_v, shared.at[sid])
    plsc.subcore_barrier()

    pltpu.sync_copy(shared.at[0, pl.ds(sid * SLICE, SLICE)], sum_v)
    for t in range(1, 16):
        pltpu.sync_copy(shared.at[t, pl.ds(sid * SLICE, SLICE)], tmp_v)

        def _accum(j, carry):
            for u in range(8):
                sl = pl.ds(j * 128 + u * 16, 16)
                sum_v[sl] = sum_v[sl] + tmp_v[sl]
            return carry

        lax.fori_loop(0, SLICE // 128, _accum, 0)

    pltpu.sync_copy(sum_v, out_hbm.at[cid, pl.ds(sid * SLICE, SLICE)])


def kernel(z, pos, batch, emb, Wp, W1, b1, W_out, b_out):
    z1 = z.astype(jnp.int32).reshape(1, N)
    post = pos.T
    w1t = W1.T
    embpt = jnp.zeros((D, D), jnp.float32).at[:, : emb.shape[0]].set(emb.T)
    b1c = b1.reshape(D, 1)
    wpt = Wp.T
    boutr = b_out.reshape(1, 1)

    y = _tc_energies(z1, post, w1t, embpt, b1c, wpt, W_out.reshape(1, D),
                     boutr)

    parts = _make_sc_segment_sum()(batch.astype(jnp.int32), y.reshape(N))
    out = (parts[0] + parts[1])[:NUM_SEGMENTS].reshape(NUM_SEGMENTS, 1)
    return out


# speedup vs baseline: 1.3313x; 1.0894x over previous
import functools

import jax
import jax.numpy as jnp
from jax import lax
from jax.experimental import pallas as pl
from jax.experimental.pallas import tpu as pltpu
from jax.experimental.pallas import tpu_sc as plsc

N = 320000
D = 128
NUM_SEGMENTS = 10000


TC_B = 6400


K_AUG = 144


def _tc_body(z_ref, post_ref, w1t_ref, embpt_ref, b1c_ref, wpt_ref, wout_ref,
             bout_ref, y_ref, lhshi_s, rhs_s):
    @pl.when(pl.program_id(0) == 0)
    def _fold_weights():
        tt = jnp.dot(w1t_ref[...], embpt_ref[...],
                     preferred_element_type=jnp.float32) + b1c_ref[...]
        mt = jnp.dot(w1t_ref[...], wpt_ref[...],
                     preferred_element_type=jnp.float32)
        tthi = tt.astype(jnp.bfloat16)
        mthi = mt.astype(jnp.bfloat16)
        lhshi_s[:, 0:D] = tthi
        lhshi_s[:, D:D + 3] = mthi
        lhshi_s[:, D + 3:K_AUG] = jnp.zeros((D, K_AUG - D - 3), jnp.bfloat16)
        rhs_s[D + 3:K_AUG, :] = jnp.zeros((K_AUG - D - 3, TC_B), jnp.bfloat16)

    z = z_ref[...]
    oht = (lax.broadcasted_iota(jnp.int32, (D, TC_B), 0) == z)
    rhs_s[0:D, :] = oht.astype(jnp.bfloat16)
    rhs_s[D:D + 3, :] = post_ref[...].astype(jnp.bfloat16)
    r = rhs_s[...]
    at = jnp.dot(lhshi_s[...], r, preferred_element_type=jnp.float32)
    ht = jnp.tanh(at)
    y_ref[...] = (jnp.dot(wout_ref[...], ht,
                          preferred_element_type=jnp.float32)
                  + bout_ref[...])


def _tc_energies(z1, post, w1t, embpt, b1c, wpt, wout, boutr, start, size):
    grid = (size // TC_B,)
    sb = start // TC_B
    return pl.pallas_call(
        _tc_body,
        grid=grid,
        in_specs=[
            pl.BlockSpec((1, TC_B), lambda i: (0, sb + i)),
            pl.BlockSpec((3, TC_B), lambda i: (0, sb + i)),
            pl.BlockSpec((D, D), lambda i: (0, 0)),
            pl.BlockSpec((D, D), lambda i: (0, 0)),
            pl.BlockSpec((D, 1), lambda i: (0, 0)),
            pl.BlockSpec((D, 3), lambda i: (0, 0)),
            pl.BlockSpec((1, D), lambda i: (0, 0)),
            pl.BlockSpec((1, 1), lambda i: (0, 0)),
        ],
        out_specs=pl.BlockSpec((1, TC_B), lambda i: (0, i)),
        out_shape=jax.ShapeDtypeStruct((1, size), jnp.float32),
        scratch_shapes=[
            pltpu.VMEM((D, K_AUG), jnp.bfloat16),
            pltpu.VMEM((K_AUG, TC_B), jnp.bfloat16),
        ],
    )(z1, post, w1t, embpt, b1c, wpt, wout, boutr)


NW = 32
CHUNKS = (192000, 128000)
S_PAD = 10240
SLICE = S_PAD // 16

@functools.lru_cache(maxsize=4)
def _make_sc_segment_sum(start, size):
    mesh = plsc.VectorSubcoreMesh(core_axis_name="c", subcore_axis_name="s")
    ch = size // NW
    body = functools.partial(_sc_segment_sum_body, start, ch)
    return pl.kernel(
        body,
        mesh=mesh,
        compiler_params=pltpu.CompilerParams(needs_layout_passes=False),
        out_type=jax.ShapeDtypeStruct((2, S_PAD), jnp.float32),
        scratch_types=[
            pltpu.VMEM((ch,), jnp.int32),
            pltpu.VMEM((ch,), jnp.float32),
            pltpu.VMEM((S_PAD,), jnp.float32),
            pltpu.VMEM((SLICE,), jnp.float32),
            pltpu.VMEM((SLICE,), jnp.float32),
            pltpu.VMEM_SHARED((16, S_PAD), jnp.float32),
        ],
    )


def _sc_segment_sum_body(start, CH, batch_hbm, y_hbm, out_hbm, b_v, y_v,
                         acc_v, sum_v, tmp_v, shared):
    cid = lax.axis_index("c")
    sid = lax.axis_index("s")
    wid = cid * 16 + sid

    zeros16 = jnp.zeros((16,), jnp.float32)

    def _zero(i, carry):
        for u in range(8):
            acc_v[pl.ds(i * 128 + u * 16, 16)] = zeros16
        return carry

    lax.fori_loop(0, S_PAD // 128, _zero, 0)

    pltpu.sync_copy(batch_hbm.at[pl.ds(start + wid * CH, CH)], b_v)
    pltpu.sync_copy(y_hbm.at[pl.ds(wid * CH, CH)], y_v)

    def _scatter(i, carry):
        for u in range(5):
            idx = b_v[pl.ds(i * 80 + u * 16, 16)]
            val = y_v[pl.ds(i * 80 + u * 16, 16)]
            plsc.addupdate_scatter(acc_v, [idx], val)
        return carry

    lax.fori_loop(0, CH // 80, _scatter, 0)

    pltpu.sync_copy(acc_v, shared.at[sid])
    plsc.subcore_barrier()

    pltpu.sync_copy(shared.at[0, pl.ds(sid * SLICE, SLICE)], sum_v)
    for t in range(1, 16):
        pltpu.sync_copy(shared.at[t, pl.ds(sid * SLICE, SLICE)], tmp_v)

        def _accum(j, carry):
            for u in range(8):
                sl = pl.ds(j * 128 + u * 16, 16)
                sum_v[sl] = sum_v[sl] + tmp_v[sl]
            return carry

        lax.fori_loop(0, SLICE // 128, _accum, 0)

    pltpu.sync_copy(sum_v, out_hbm.at[cid, pl.ds(sid * SLICE, SLICE)])


def kernel(z, pos, batch, emb, Wp, W1, b1, W_out, b_out):
    z1 = z.astype(jnp.int32).reshape(1, N)
    post = pos.T
    w1t = W1.T
    embpt = jnp.zeros((D, D), jnp.float32).at[:, : emb.shape[0]].set(emb.T)
    b1c = b1.reshape(D, 1)
    wpt = Wp.T
    boutr = b_out.reshape(1, 1)

    batch_i = batch.astype(jnp.int32)
    woutr = W_out.reshape(1, D)

    acc = None
    start = 0
    for size in CHUNKS:
        yc = _tc_energies(z1, post, w1t, embpt, b1c, wpt, woutr, boutr,
                          start, size)
        parts = _make_sc_segment_sum(start, size)(batch_i, yc.reshape(size))
        p = parts[0] + parts[1]
        acc = p if acc is None else acc + p
        start += size
    out = acc[:NUM_SEGMENTS].reshape(NUM_SEGMENTS, 1)
    return out


# speedup vs baseline: 1.3961x; 1.0487x over previous
import functools

import jax
import jax.numpy as jnp
from jax import lax
from jax.experimental import pallas as pl
from jax.experimental.pallas import tpu as pltpu
from jax.experimental.pallas import tpu_sc as plsc

N = 320000
D = 128
NUM_SEGMENTS = 10000


TC_B = 6400


K_AUG = 144


def _tc_body(z_ref, post_ref, w1t_ref, embpt_ref, b1c_ref, wpt_ref, wout_ref,
             bout_ref, y_ref, lhshi_s, rhs_s):
    @pl.when(pl.program_id(0) == 0)
    def _fold_weights():
        tt = jnp.dot(w1t_ref[...], embpt_ref[...],
                     preferred_element_type=jnp.float32) + b1c_ref[...]
        mt = jnp.dot(w1t_ref[...], wpt_ref[...],
                     preferred_element_type=jnp.float32)
        tthi = tt.astype(jnp.bfloat16)
        mthi = mt.astype(jnp.bfloat16)
        lhshi_s[:, 0:D] = tthi
        lhshi_s[:, D:D + 3] = mthi
        lhshi_s[:, D + 3:K_AUG] = jnp.zeros((D, K_AUG - D - 3), jnp.bfloat16)
        rhs_s[D + 3:K_AUG, :] = jnp.zeros((K_AUG - D - 3, TC_B), jnp.bfloat16)

    z = z_ref[...].reshape(1, TC_B)
    oht = (lax.broadcasted_iota(jnp.int32, (D, TC_B), 0) == z)
    rhs_s[0:D, :] = oht.astype(jnp.bfloat16)
    rhs_s[D:D + 3, :] = post_ref[...].astype(jnp.bfloat16)
    r = rhs_s[...]
    at = jnp.dot(lhshi_s[...], r, preferred_element_type=jnp.float32)
    ht = jnp.tanh(at)
    yrow = (jnp.dot(wout_ref[...], ht, preferred_element_type=jnp.float32)
            + bout_ref[...])
    y_ref[...] = yrow.reshape(1, TC_B // 128, 128)


def _tc_energies(z1, post, w1t, embpt, b1c, wpt, wout, boutr, start, size):
    grid = (size // TC_B,)
    sb = start // TC_B
    return pl.pallas_call(
        _tc_body,
        grid=grid,
        in_specs=[
            pl.BlockSpec((1, TC_B // 128, 128), lambda i: (sb + i, 0, 0)),
            pl.BlockSpec((3, TC_B), lambda i: (0, sb + i)),
            pl.BlockSpec((D, D), lambda i: (0, 0)),
            pl.BlockSpec((D, D), lambda i: (0, 0)),
            pl.BlockSpec((D, 1), lambda i: (0, 0)),
            pl.BlockSpec((D, 3), lambda i: (0, 0)),
            pl.BlockSpec((1, D), lambda i: (0, 0)),
            pl.BlockSpec((1, 1), lambda i: (0, 0)),
        ],
        out_specs=pl.BlockSpec((1, TC_B // 128, 128), lambda i: (i, 0, 0)),
        out_shape=jax.ShapeDtypeStruct(
            (size // TC_B, TC_B // 128, 128), jnp.float32),
        scratch_shapes=[
            pltpu.VMEM((D, K_AUG), jnp.bfloat16),
            pltpu.VMEM((K_AUG, TC_B), jnp.bfloat16),
        ],
    )(z1, post, w1t, embpt, b1c, wpt, wout, boutr)


NW = 32
CHUNKS = (192000, 128000)
S_PAD = 10240
SLICE = S_PAD // 16

@functools.lru_cache(maxsize=4)
def _make_sc_segment_sum(start, size):
    mesh = plsc.VectorSubcoreMesh(core_axis_name="c", subcore_axis_name="s")
    ch = size // NW
    body = functools.partial(_sc_segment_sum_body, start, ch)
    return pl.kernel(
        body,
        mesh=mesh,
        compiler_params=pltpu.CompilerParams(needs_layout_passes=False),
        out_type=jax.ShapeDtypeStruct((2, S_PAD), jnp.float32),
        scratch_types=[
            pltpu.VMEM((ch,), jnp.int32),
            pltpu.VMEM((ch,), jnp.float32),
            pltpu.VMEM((S_PAD,), jnp.float32),
            pltpu.VMEM((SLICE,), jnp.float32),
            pltpu.VMEM((SLICE,), jnp.float32),
            pltpu.VMEM_SHARED((16, S_PAD), jnp.float32),
        ],
    )


def _sc_segment_sum_body(start, CH, batch_hbm, y_hbm, out_hbm, b_v, y_v,
                         acc_v, sum_v, tmp_v, shared):
    cid = lax.axis_index("c")
    sid = lax.axis_index("s")
    wid = cid * 16 + sid

    zeros16 = jnp.zeros((16,), jnp.float32)

    def _zero(i, carry):
        for u in range(8):
            acc_v[pl.ds(i * 128 + u * 16, 16)] = zeros16
        return carry

    lax.fori_loop(0, S_PAD // 128, _zero, 0)

    pltpu.sync_copy(batch_hbm.at[pl.ds(start + wid * CH, CH)], b_v)
    pltpu.sync_copy(y_hbm.at[pl.ds(wid * CH, CH)], y_v)

    def _scatter(i, carry):
        for u in range(5):
            idx = b_v[pl.ds(i * 80 + u * 16, 16)]
            val = y_v[pl.ds(i * 80 + u * 16, 16)]
            plsc.addupdate_scatter(acc_v, [idx], val)
        return carry

    lax.fori_loop(0, CH // 80, _scatter, 0)

    pltpu.sync_copy(acc_v, shared.at[sid])
    plsc.subcore_barrier()

    pltpu.sync_copy(shared.at[0, pl.ds(sid * SLICE, SLICE)], sum_v)
    for t in range(1, 16):
        pltpu.sync_copy(shared.at[t, pl.ds(sid * SLICE, SLICE)], tmp_v)

        def _accum(j, carry):
            for u in range(8):
                sl = pl.ds(j * 128 + u * 16, 16)
                sum_v[sl] = sum_v[sl] + tmp_v[sl]
            return carry

        lax.fori_loop(0, SLICE // 128, _accum, 0)

    pltpu.sync_copy(sum_v, out_hbm.at[cid, pl.ds(sid * SLICE, SLICE)])


def kernel(z, pos, batch, emb, Wp, W1, b1, W_out, b_out):
    z1 = z.astype(jnp.int32).reshape(N // TC_B, TC_B // 128, 128)
    post = pos.T
    w1t = W1.T
    embpt = jnp.zeros((D, D), jnp.float32).at[:, : emb.shape[0]].set(emb.T)
    b1c = b1.reshape(D, 1)
    wpt = Wp.T
    boutr = b_out.reshape(1, 1)

    batch_i = batch.astype(jnp.int32)
    woutr = W_out.reshape(1, D)

    acc = None
    start = 0
    for size in CHUNKS:
        yc = _tc_energies(z1, post, w1t, embpt, b1c, wpt, woutr, boutr,
                          start, size)
        parts = _make_sc_segment_sum(start, size)(batch_i, yc.reshape(size))
        p = parts[0] + parts[1]
        acc = p if acc is None else acc + p
        start += size
    out = acc[:NUM_SEGMENTS].reshape(NUM_SEGMENTS, 1)
    return out


# speedup vs baseline: 1.4622x; 1.0473x over previous
import functools

import jax
import jax.numpy as jnp
from jax import lax
from jax.experimental import pallas as pl
from jax.experimental.pallas import tpu as pltpu
from jax.experimental.pallas import tpu_sc as plsc

N = 320000
D = 128
NUM_SEGMENTS = 10000


TC_B = 6400


K_AUG = 144


def _tc_body(z_ref, post_ref, w1t_ref, embpt_ref, b1c_ref, wpt_ref, wout_ref,
             bout_ref, y_ref, lhshi_s, rhs_s):
    @pl.when(pl.program_id(0) == 0)
    def _fold_weights():
        tt = jnp.dot(w1t_ref[...], embpt_ref[...],
                     preferred_element_type=jnp.float32) + b1c_ref[...]
        mt = jnp.dot(w1t_ref[...], wpt_ref[...],
                     preferred_element_type=jnp.float32)
        tthi = tt.astype(jnp.bfloat16)
        mthi = mt.astype(jnp.bfloat16)
        lhshi_s[:, 0:D] = tthi
        lhshi_s[:, D:D + 3] = mthi
        lhshi_s[:, D + 3:K_AUG] = jnp.zeros((D, K_AUG - D - 3), jnp.bfloat16)
        rhs_s[D + 3:K_AUG, :] = jnp.zeros((K_AUG - D - 3, TC_B), jnp.bfloat16)

    z = z_ref[...].reshape(1, TC_B)
    oht = (lax.broadcasted_iota(jnp.int32, (D, TC_B), 0) == z)
    rhs_s[0:D, :] = oht.astype(jnp.bfloat16)
    rhs_s[D:D + 3, :] = post_ref[...].astype(jnp.bfloat16)
    r = rhs_s[...]
    at = jnp.dot(lhshi_s[...], r, preferred_element_type=jnp.float32)
    ht = jnp.tanh(at)
    yrow = (jnp.dot(wout_ref[...], ht, preferred_element_type=jnp.float32)
            + bout_ref[...])
    y_ref[...] = yrow.reshape(1, TC_B // 128, 128)


def _tc_energies(z1, post, w1t, embpt, b1c, wpt, wout, boutr, start, size):
    grid = (size // TC_B,)
    sb = start // TC_B
    return pl.pallas_call(
        _tc_body,
        grid=grid,
        in_specs=[
            pl.BlockSpec((1, TC_B // 128, 128), lambda i: (sb + i, 0, 0)),
            pl.BlockSpec((3, TC_B), lambda i: (0, sb + i)),
            pl.BlockSpec((D, D), lambda i: (0, 0)),
            pl.BlockSpec((D, D), lambda i: (0, 0)),
            pl.BlockSpec((D, 1), lambda i: (0, 0)),
            pl.BlockSpec((D, 3), lambda i: (0, 0)),
            pl.BlockSpec((1, D), lambda i: (0, 0)),
            pl.BlockSpec((1, 1), lambda i: (0, 0)),
        ],
        out_specs=pl.BlockSpec((1, TC_B // 128, 128), lambda i: (i, 0, 0)),
        out_shape=jax.ShapeDtypeStruct(
            (size // TC_B, TC_B // 128, 128), jnp.float32),
        scratch_shapes=[
            pltpu.VMEM((D, K_AUG), jnp.bfloat16),
            pltpu.VMEM((K_AUG, TC_B), jnp.bfloat16),
        ],
    )(z1, post, w1t, embpt, b1c, wpt, wout, boutr)


NW = 32
CHUNKS = (192000, 128000)
S_PAD = 10240
SLICE = S_PAD // 16

@functools.lru_cache(maxsize=4)
def _make_sc_segment_sum(start, size):
    mesh = plsc.VectorSubcoreMesh(core_axis_name="c", subcore_axis_name="s")
    ch = size // NW
    body = functools.partial(_sc_segment_sum_body, start, ch)
    return pl.kernel(
        body,
        mesh=mesh,
        compiler_params=pltpu.CompilerParams(needs_layout_passes=False),
        out_type=jax.ShapeDtypeStruct((2, S_PAD), jnp.float32),
        scratch_types=[
            pltpu.VMEM((ch,), jnp.int32),
            pltpu.VMEM((ch,), jnp.float32),
            pltpu.VMEM((S_PAD,), jnp.float32),
            pltpu.VMEM((SLICE,), jnp.float32),
            pltpu.VMEM((SLICE,), jnp.float32),
            pltpu.VMEM_SHARED((16, S_PAD), jnp.float32),
        ],
    )


def _sc_segment_sum_body(start, CH, batch_hbm, y_hbm, out_hbm, b_v, y_v,
                         acc_v, sum_v, tmp_v, shared):
    cid = lax.axis_index("c")
    sid = lax.axis_index("s")
    wid = cid * 16 + sid

    zeros16 = jnp.zeros((16,), jnp.float32)

    def _zero(i, carry):
        for u in range(8):
            acc_v[pl.ds(i * 128 + u * 16, 16)] = zeros16
        return carry

    lax.fori_loop(0, S_PAD // 128, _zero, 0)

    pltpu.sync_copy(batch_hbm.at[pl.ds(start + wid * CH, CH)], b_v)
    pltpu.sync_copy(y_hbm.at[pl.ds(wid * CH, CH)], y_v)

    lane_base = lax.iota(jnp.int32, 16) * (CH // 16)

    def _scatter(i, carry):
        for u in range(5):
            pos16 = lane_base + (i * 5 + u)
            idx = plsc.load_gather(b_v, [pos16])
            val = plsc.load_gather(y_v, [pos16])
            plsc.addupdate_scatter(acc_v, [idx], val)
        return carry

    lax.fori_loop(0, CH // 80, _scatter, 0)

    pltpu.sync_copy(acc_v, shared.at[sid])
    plsc.subcore_barrier()

    pltpu.sync_copy(shared.at[0, pl.ds(sid * SLICE, SLICE)], sum_v)
    for t in range(1, 16):
        pltpu.sync_copy(shared.at[t, pl.ds(sid * SLICE, SLICE)], tmp_v)

        def _accum(j, carry):
            for u in range(8):
                sl = pl.ds(j * 128 + u * 16, 16)
                sum_v[sl] = sum_v[sl] + tmp_v[sl]
            return carry

        lax.fori_loop(0, SLICE // 128, _accum, 0)

    pltpu.sync_copy(sum_v, out_hbm.at[cid, pl.ds(sid * SLICE, SLICE)])


def kernel(z, pos, batch, emb, Wp, W1, b1, W_out, b_out):
    z1 = z.astype(jnp.int32).reshape(N // TC_B, TC_B // 128, 128)
    post = pos.T
    w1t = W1.T
    embpt = jnp.zeros((D, D), jnp.float32).at[:, : emb.shape[0]].set(emb.T)
    b1c = b1.reshape(D, 1)
    wpt = Wp.T
    boutr = b_out.reshape(1, 1)

    batch_i = batch.astype(jnp.int32)
    woutr = W_out.reshape(1, D)

    acc = None
    start = 0
    for size in CHUNKS:
        yc = _tc_energies(z1, post, w1t, embpt, b1c, wpt, woutr, boutr,
                          start, size)
        parts = _make_sc_segment_sum(start, size)(batch_i, yc.reshape(size))
        p = parts[0] + parts[1]
        acc = p if acc is None else acc + p
        start += size
    out = acc[:NUM_SEGMENTS].reshape(NUM_SEGMENTS, 1)
    return out


# speedup vs baseline: 1.4713x; 1.0062x over previous
import functools

import jax
import jax.numpy as jnp
from jax import lax
from jax.experimental import pallas as pl
from jax.experimental.pallas import tpu as pltpu
from jax.experimental.pallas import tpu_sc as plsc

N = 320000
D = 128
NUM_SEGMENTS = 10000


TC_B = 6400


K_AUG = 144


def _tc_body(z_ref, post_ref, w1t_ref, embpt_ref, b1c_ref, wpt_ref, wout_ref,
             bout_ref, y_ref, lhshi_s, rhs_s):
    @pl.when(pl.program_id(0) == 0)
    def _fold_weights():
        tt = jnp.dot(w1t_ref[...], embpt_ref[...],
                     preferred_element_type=jnp.float32) + b1c_ref[...]
        mt = jnp.dot(w1t_ref[...], wpt_ref[...],
                     preferred_element_type=jnp.float32)
        tthi = tt.astype(jnp.bfloat16)
        mthi = mt.astype(jnp.bfloat16)
        lhshi_s[:, 0:D] = tthi
        lhshi_s[:, D:D + 3] = mthi
        lhshi_s[:, D + 3:K_AUG] = jnp.zeros((D, K_AUG - D - 3), jnp.bfloat16)
        rhs_s[D + 3:K_AUG, :] = jnp.zeros((K_AUG - D - 3, TC_B), jnp.bfloat16)

    z = z_ref[...].reshape(1, TC_B)
    oht = (lax.broadcasted_iota(jnp.int32, (D, TC_B), 0) == z)
    rhs_s[0:D, :] = oht.astype(jnp.bfloat16)
    rhs_s[D:D + 3, :] = post_ref[...].astype(jnp.bfloat16)
    r = rhs_s[...]
    at = jnp.dot(lhshi_s[...], r, preferred_element_type=jnp.float32)
    ht = jnp.tanh(at)
    yrow = (jnp.dot(wout_ref[...], ht, preferred_element_type=jnp.float32)
            + bout_ref[...])
    y_ref[...] = yrow.reshape(1, TC_B // 128, 128)


def _tc_energies(z1, post, w1t, embpt, b1c, wpt, wout, boutr, start, size):
    grid = (size // TC_B,)
    sb = start // TC_B
    return pl.pallas_call(
        _tc_body,
        grid=grid,
        in_specs=[
            pl.BlockSpec((1, TC_B // 128, 128), lambda i: (sb + i, 0, 0)),
            pl.BlockSpec((3, TC_B), lambda i: (0, sb + i)),
            pl.BlockSpec((D, D), lambda i: (0, 0)),
            pl.BlockSpec((D, D), lambda i: (0, 0)),
            pl.BlockSpec((D, 1), lambda i: (0, 0)),
            pl.BlockSpec((D, 3), lambda i: (0, 0)),
            pl.BlockSpec((1, D), lambda i: (0, 0)),
            pl.BlockSpec((1, 1), lambda i: (0, 0)),
        ],
        out_specs=pl.BlockSpec((1, TC_B // 128, 128), lambda i: (i, 0, 0)),
        out_shape=jax.ShapeDtypeStruct(
            (size // TC_B, TC_B // 128, 128), jnp.float32),
        scratch_shapes=[
            pltpu.VMEM((D, K_AUG), jnp.bfloat16),
            pltpu.VMEM((K_AUG, TC_B), jnp.bfloat16),
        ],
    )(z1, post, w1t, embpt, b1c, wpt, wout, boutr)


NW = 32
CHUNKS = (128000, 102400, 89600)
S_PAD = 10240
SLICE = S_PAD // 16

@functools.lru_cache(maxsize=4)
def _make_sc_segment_sum(start, size):
    mesh = plsc.VectorSubcoreMesh(core_axis_name="c", subcore_axis_name="s")
    ch = size // NW
    body = functools.partial(_sc_segment_sum_body, start, ch)
    return pl.kernel(
        body,
        mesh=mesh,
        compiler_params=pltpu.CompilerParams(needs_layout_passes=False),
        out_type=jax.ShapeDtypeStruct((2, S_PAD), jnp.float32),
        scratch_types=[
            pltpu.VMEM((ch,), jnp.int32),
            pltpu.VMEM((ch,), jnp.float32),
            pltpu.VMEM((S_PAD,), jnp.float32),
            pltpu.VMEM((SLICE,), jnp.float32),
            pltpu.VMEM((SLICE,), jnp.float32),
            pltpu.VMEM_SHARED((16, S_PAD), jnp.float32),
        ],
    )


def _sc_segment_sum_body(start, CH, batch_hbm, y_hbm, out_hbm, b_v, y_v,
                         acc_v, sum_v, tmp_v, shared):
    cid = lax.axis_index("c")
    sid = lax.axis_index("s")
    wid = cid * 16 + sid

    zeros16 = jnp.zeros((16,), jnp.float32)

    def _zero(i, carry):
        for u in range(8):
            acc_v[pl.ds(i * 128 + u * 16, 16)] = zeros16
        return carry

    lax.fori_loop(0, S_PAD // 128, _zero, 0)

    pltpu.sync_copy(batch_hbm.at[pl.ds(start + wid * CH, CH)], b_v)
    pltpu.sync_copy(y_hbm.at[pl.ds(wid * CH, CH)], y_v)

    lane_base = lax.iota(jnp.int32, 16) * (CH // 16)

    def _scatter(i, carry):
        for u in range(5):
            pos16 = lane_base + (i * 5 + u)
            idx = plsc.load_gather(b_v, [pos16])
            val = plsc.load_gather(y_v, [pos16])
            plsc.addupdate_scatter(acc_v, [idx], val)
        return carry

    lax.fori_loop(0, CH // 80, _scatter, 0)

    pltpu.sync_copy(acc_v, shared.at[sid])
    plsc.subcore_barrier()

    pltpu.sync_copy(shared.at[0, pl.ds(sid * SLICE, SLICE)], sum_v)
    for t in range(1, 16):
        pltpu.sync_copy(shared.at[t, pl.ds(sid * SLICE, SLICE)], tmp_v)

        def _accum(j, carry):
            for u in range(8):
                sl = pl.ds(j * 128 + u * 16, 16)
                sum_v[sl] = sum_v[sl] + tmp_v[sl]
            return carry

        lax.fori_loop(0, SLICE // 128, _accum, 0)

    pltpu.sync_copy(sum_v, out_hbm.at[cid, pl.ds(sid * SLICE, SLICE)])


def kernel(z, pos, batch, emb, Wp, W1, b1, W_out, b_out):
    z1 = z.astype(jnp.int32).reshape(N // TC_B, TC_B // 128, 128)
    post = pos.T
    w1t = W1.T
    embpt = jnp.zeros((D, D), jnp.float32).at[:, : emb.shape[0]].set(emb.T)
    b1c = b1.reshape(D, 1)
    wpt = Wp.T
    boutr = b_out.reshape(1, 1)

    batch_i = batch.astype(jnp.int32)
    woutr = W_out.reshape(1, D)

    acc = None
    start = 0
    for size in CHUNKS:
        yc = _tc_energies(z1, post, w1t, embpt, b1c, wpt, woutr, boutr,
                          start, size)
        parts = _make_sc_segment_sum(start, size)(batch_i, yc.reshape(size))
        p = parts[0] + parts[1]
        acc = p if acc is None else acc + p
        start += size
    out = acc[:NUM_SEGMENTS].reshape(NUM_SEGMENTS, 1)
    return out
